# Initial kernel scaffold; baseline (speedup 1.0000x reference)
import jax
import jax.numpy as jnp
from jax.experimental import pallas as pl


def kernel(x, edge_index, edge_weight, batch, Wrel, brel, Wroot):
    raise NotImplementedError("write your pallas kernel here")


# speedup vs baseline: 4.2472x; 4.2472x over previous
import functools

import jax
import jax.numpy as jnp
from jax import lax
from jax.experimental import pallas as pl
from jax.experimental.pallas import tpu as pltpu
from jax.experimental.pallas import tpu_sc as plsc

N = 10000
E = 320000
H = 128
G = 128

NC = 2
NS = 16
NW = NC * NS
EPW = E // NW
K = 80
NCHUNK = EPW // K
WR = 624
XR = N - NS * WR

_mesh = plsc.VectorSubcoreMesh(
    core_axis_name="c", subcore_axis_name="s", num_cores=NC, num_subcores=NS
)


@functools.partial(
    pl.kernel,
    out_type=jax.ShapeDtypeStruct((NC, N, H), jnp.float32),
    mesh=_mesh,
    scratch_types=[
        pltpu.VMEM((2, K), jnp.int32),
        pltpu.VMEM((EPW,), jnp.float32),
        pltpu.VMEM((K, H), jnp.float32),
        pltpu.VMEM_SHARED((N, H), jnp.float32),
        pltpu.SemaphoreType.DMA,
    ],
    compiler_params=pltpu.CompilerParams(needs_layout_passes=False),
)
def _sc_agg(h_hbm, sd_hbm, w_hbm, out_hbm,
            sd_v, w_v, rows_v, acc_sh, sem):
    cid = lax.axis_index("c")
    sid = lax.axis_index("s")
    wid = cid * NS + sid

    zvec = jnp.zeros((16,), jnp.float32)

    def zrow(i, _):
        for c in range(H // 16):
            rows_v[i, pl.ds(c * 16, 16)] = zvec
        return 0

    lax.fori_loop(0, K, zrow, 0)
    zbase = sid * WR
    for j in range(WR // K):
        pltpu.sync_copy(rows_v, acc_sh.at[pl.ds(zbase + j * K, K)])
    zrem = WR - (WR // K) * K
    if zrem:
        pltpu.sync_copy(rows_v.at[pl.ds(0, zrem)],
                        acc_sh.at[pl.ds(zbase + (WR // K) * K, zrem)])

    @pl.when(sid == NS - 1)
    def _ztail():
        pltpu.sync_copy(rows_v.at[pl.ds(0, XR)], acc_sh.at[pl.ds(NS * WR, XR)])

    plsc.subcore_barrier()

    pltpu.sync_copy(w_hbm.at[wid], w_v)

    def chunk(g, _):
        pltpu.sync_copy(sd_hbm.at[wid, g], sd_v)
        pltpu.async_copy(h_hbm.at[sd_v.at[0]], rows_v, sem).wait()

        def edge(i, _):
            ii = jnp.full((16,), g * K + i, jnp.int32)
            wv = plsc.load_gather(w_v, [ii])
            for c in range(H // 16):
                sl = pl.ds(c * 16, 16)
                rows_v[i, sl] = rows_v[i, sl] * wv
            return 0

        lax.fori_loop(0, K, edge, 0)

        pltpu.sync_copy(rows_v, acc_sh.at[sd_v.at[1]], add=True)
        return 0

    lax.fori_loop(0, NCHUNK, chunk, 0)
    plsc.subcore_barrier()

    obase = sid * WR
    pltpu.sync_copy(acc_sh.at[pl.ds(obase, WR)],
                    out_hbm.at[cid, pl.ds(obase, WR)])

    @pl.when(sid == NS - 1)
    def _otail():
        pltpu.sync_copy(acc_sh.at[pl.ds(NS * WR, XR)],
                        out_hbm.at[cid, pl.ds(NS * WR, XR)])


R = 1000
NB = N // R


def _tc_layer_body(h_ref, agg_ref, wrel_ref, wroot_ref, b_ref, out_ref, *, relu):
    agg = agg_ref[0] + agg_ref[1]
    out = (jnp.dot(agg, wrel_ref[...], preferred_element_type=jnp.float32)
           + jnp.dot(h_ref[...], wroot_ref[...], preferred_element_type=jnp.float32)
           + b_ref[...])
    if relu:
        out = jnp.maximum(out, 0.0)
    out_ref[...] = out


def _tc_layer(h, agg2, wrelT, wrootT, b, relu):
    return pl.pallas_call(
        functools.partial(_tc_layer_body, relu=relu),
        grid=(NB,),
        in_specs=[
            pl.BlockSpec((R, H), lambda i: (i, 0)),
            pl.BlockSpec((NC, R, H), lambda i: (0, i, 0)),
            pl.BlockSpec((H, H), lambda i: (0, 0)),
            pl.BlockSpec((H, H), lambda i: (0, 0)),
            pl.BlockSpec((1, H), lambda i: (0, 0)),
        ],
        out_specs=pl.BlockSpec((R, H), lambda i: (i, 0)),
        out_shape=jax.ShapeDtypeStruct((N, H), jnp.float32),
    )(h, agg2, wrelT, wrootT, b)


def _tc_final_body(h_ref, agg_ref, wrel_ref, wroot_ref, b_ref, batch_ref,
                   out_ref, sums_ref, cnt_ref):
    i = pl.program_id(0)

    @pl.when(i == 0)
    def _init():
        sums_ref[...] = jnp.zeros((G, H), jnp.float32)
        cnt_ref[...] = jnp.zeros((G, H), jnp.float32)

    agg = agg_ref[0] + agg_ref[1]
    h3 = (jnp.dot(agg, wrel_ref[...], preferred_element_type=jnp.float32)
          + jnp.dot(h_ref[...], wroot_ref[...], preferred_element_type=jnp.float32)
          + b_ref[...])
    bv = batch_ref[0, 0, :]
    seg = lax.broadcasted_iota(jnp.int32, (G, R), 0)
    onehot = (seg == bv[None, :]).astype(jnp.float32)
    sums_ref[...] += jnp.dot(onehot, h3, preferred_element_type=jnp.float32)
    cnt_ref[...] += jnp.dot(onehot, jnp.ones((R, H), jnp.float32),
                            preferred_element_type=jnp.float32)

    @pl.when(i == NB - 1)
    def _fin():
        out_ref[...] = sums_ref[...] / jnp.maximum(cnt_ref[...], 1.0)


def _tc_final(h, agg2, wrelT, wrootT, b, batch_f):
    return pl.pallas_call(
        _tc_final_body,
        grid=(NB,),
        in_specs=[
            pl.BlockSpec((R, H), lambda i: (i, 0)),
            pl.BlockSpec((NC, R, H), lambda i: (0, i, 0)),
            pl.BlockSpec((H, H), lambda i: (0, 0)),
            pl.BlockSpec((H, H), lambda i: (0, 0)),
            pl.BlockSpec((1, H), lambda i: (0, 0)),
            pl.BlockSpec((1, 1, R), lambda i: (i, 0, 0)),
        ],
        out_specs=pl.BlockSpec((G, H), lambda i: (0, 0)),
        out_shape=jax.ShapeDtypeStruct((G, H), jnp.float32),
        scratch_shapes=[
            pltpu.VMEM((G, H), jnp.float32),
            pltpu.VMEM((G, H), jnp.float32),
        ],
    )(h, agg2, wrelT, wrootT, b, batch_f)


def kernel(x, edge_index, edge_weight, batch, Wrel, brel, Wroot):
    src3 = edge_index[0].reshape(NW, NCHUNK, K)
    dst3 = edge_index[1].reshape(NW, NCHUNK, K)
    sd4 = jnp.stack([src3, dst3], axis=2)
    w3 = edge_weight.reshape(NW, EPW)
    batch_f = batch.reshape(NB, 1, R)

    h = x
    out = None
    for i in range(Wrel.shape[0]):
        agg2 = _sc_agg(h, sd4, w3)
        wrelT = Wrel[i].T
        wrootT = Wroot[i].T
        b = brel[i][None, :]
        if i < Wrel.shape[0] - 1:
            h = _tc_layer(h, agg2, wrelT, wrootT, b, relu=True)
        else:
            out = _tc_final(h, agg2, wrelT, wrootT, b, batch_f)
    return out
